# baseline (speedup 1.0000x reference)
import jax
import jax.numpy as jnp
from jax import lax
from jax.experimental import pallas as pl
from jax.experimental.pallas import tpu as pltpu
from jax.experimental.pallas import tpu_sc as plsc

NUM_EMBEDDINGS = 1000000
EMBED_DIM = 64
SEQ = 4096
NUM_TOKENS = 200

_INFO = plsc.get_sparse_core_info()
NC = _INFO.num_cores
NS = _INFO.num_subcores
NW = NC * NS
SPW = SEQ // NW
SPLIT0 = 104
SPLIT1 = NUM_TOKENS - SPLIT0
NBUF = 4


def _body(x_hbm, table_hbm, out_hbm, idx_v,
          buf0, buf1, buf2, buf3,
          g0, g1, g2, g3, s0, s1, s2, s3):
    bufs = (buf0, buf1, buf2, buf3)
    gsems = (g0, g1, g2, g3)
    ssems = (s0, s1, s2, s3)
    wid = lax.axis_index("s") * NC + lax.axis_index("c")
    sbase = wid * SPW
    rbase = sbase * NUM_TOKENS
    pltpu.sync_copy(x_hbm.at[pl.ds(sbase, SPW)], idx_v)

    def fire_gather(i, buf, gsem):
        pltpu.async_copy(table_hbm.at[idx_v.at[i, pl.ds(0, SPLIT0)]],
                         buf.at[pl.ds(0, SPLIT0)], gsem)
        pltpu.async_copy(table_hbm.at[idx_v.at[i, pl.ds(SPLIT0, SPLIT1)]],
                         buf.at[pl.ds(SPLIT0, SPLIT1)], gsem)

    def drain(buf, sem):
        pltpu.make_async_copy(
            table_hbm.at[pl.ds(0, NUM_TOKENS)], buf, sem).wait()

    for b in range(NBUF):
        fire_gather(b, bufs[b], gsems[b])

    @pl.loop(0, SPW, step=NBUF)
    def cycle(i):
        for b in range(NBUF):
            row = i + b
            drain(bufs[b], gsems[b])
            pltpu.async_copy(
                bufs[b],
                out_hbm.at[pl.ds(rbase + row * NUM_TOKENS, NUM_TOKENS)],
                ssems[b])
            drain(bufs[b], ssems[b])

            @pl.when(row + NBUF < SPW)
            def _():
                fire_gather(row + NBUF, bufs[b], gsems[b])


_sc_gather = pl.kernel(
    _body,
    out_type=jax.ShapeDtypeStruct((SEQ * NUM_TOKENS, EMBED_DIM), jnp.float32),
    mesh=plsc.VectorSubcoreMesh(core_axis_name="c", subcore_axis_name="s"),
    scratch_types=(
        [pltpu.VMEM((SPW, NUM_TOKENS), jnp.int32)]
        + [pltpu.VMEM((NUM_TOKENS, EMBED_DIM), jnp.float32)] * NBUF
        + [pltpu.SemaphoreType.DMA] * (2 * NBUF)
    ),
    compiler_params=pltpu.CompilerParams(
        use_tc_tiling_on_sc=False, needs_layout_passes=True),
)


@jax.jit
def kernel(x, embed_table):
    table_lin = jnp.take(embed_table, jnp.arange(NUM_EMBEDDINGS,
                                                 dtype=jnp.int32), axis=0)
    out2d = _sc_gather(x.astype(jnp.int32), table_lin)
    out_lin = jnp.take(out2d, jnp.arange(SEQ * NUM_TOKENS, dtype=jnp.int32),
                       axis=0)
    return out_lin.reshape(SEQ, NUM_TOKENS, EMBED_DIM)


# speedup vs baseline: 2.5275x; 2.5275x over previous
import jax
import jax.numpy as jnp
from jax import lax
from jax.experimental import pallas as pl
from jax.experimental.pallas import tpu as pltpu
from jax.experimental.pallas import tpu_sc as plsc
from jax.experimental.layout import Format, Layout, with_layout_constraint

NUM_EMBEDDINGS = 1000000
EMBED_DIM = 64
SEQ = 4096
NUM_TOKENS = 200

_INFO = plsc.get_sparse_core_info()
NC = _INFO.num_cores
NS = _INFO.num_subcores
NW = NC * NS
SPW = SEQ // NW
SPLIT0 = 104
SPLIT1 = NUM_TOKENS - SPLIT0
NBUF = 4


def _body(x_hbm, table_hbm, out_hbm, idx_v,
          buf0, buf1, buf2, buf3,
          g0, g1, g2, g3, s0, s1, s2, s3):
    bufs = (buf0, buf1, buf2, buf3)
    gsems = (g0, g1, g2, g3)
    ssems = (s0, s1, s2, s3)

    wid = lax.axis_index("s") * NC + lax.axis_index("c")
    sbase = wid * SPW
    rbase = sbase * NUM_TOKENS
    pltpu.sync_copy(x_hbm.at[pl.ds(sbase, SPW)], idx_v)

    def fire_gather(i, buf, gsem):
        pltpu.async_copy(table_hbm.at[idx_v.at[i, pl.ds(0, SPLIT0)]],
                         buf.at[pl.ds(0, SPLIT0)], gsem)
        pltpu.async_copy(table_hbm.at[idx_v.at[i, pl.ds(SPLIT0, SPLIT1)]],
                         buf.at[pl.ds(SPLIT0, SPLIT1)], gsem)

    def drain(buf, sem):
        pltpu.make_async_copy(
            table_hbm.at[pl.ds(0, NUM_TOKENS)], buf, sem).wait()

    for b in range(NBUF):
        fire_gather(b, bufs[b], gsems[b])

    @pl.loop(0, SPW, step=NBUF)
    def cycle(i):
        for b in range(NBUF):
            row = i + b
            drain(bufs[b], gsems[b])
            pltpu.async_copy(
                bufs[b],
                out_hbm.at[pl.ds(rbase + row * NUM_TOKENS, NUM_TOKENS)],
                ssems[b])
            drain(bufs[b], ssems[b])

            @pl.when(row + NBUF < SPW)
            def _():
                fire_gather(row + NBUF, bufs[b], gsems[b])


_sc_gather = pl.kernel(
    _body,
    out_type=jax.ShapeDtypeStruct((SEQ * NUM_TOKENS, EMBED_DIM), jnp.float32),
    mesh=plsc.VectorSubcoreMesh(core_axis_name="c", subcore_axis_name="s"),
    scratch_types=(
        [pltpu.VMEM((SPW, NUM_TOKENS), jnp.int32)]
        + [pltpu.VMEM((NUM_TOKENS, EMBED_DIM), jnp.float32)] * NBUF
        + [pltpu.SemaphoreType.DMA] * (2 * NBUF)
    ),
    compiler_params=pltpu.CompilerParams(use_tc_tiling_on_sc=False),
)

@jax.jit
def kernel(x, embed_table):
    fmt = Layout(major_to_minor=(0, 1), tiling=((16,),))
    table_lin = with_layout_constraint(embed_table, fmt)
    out2d = _sc_gather(x.astype(jnp.int32), table_lin)
    out2d = with_layout_constraint(out2d, fmt)
    return out2d.reshape(SEQ, NUM_TOKENS, EMBED_DIM)
